# baseline (speedup 1.0000x reference)
import functools

import jax
import jax.numpy as jnp
from jax.experimental import pallas as pl
from jax.experimental.pallas import tpu as pltpu
from jax.experimental.pallas import tpu_sc as plsc

_EMBED = 64
_NMETA = 16
_CTX = 128
_P = 50

_GATHER_WINDOW = 128
_BBLK = 256


def _sc_gather(emb_table, flat_ids):
    n = flat_ids.shape[1]
    mesh = plsc.VectorSubcoreMesh(core_axis_name="c", subcore_axis_name="s")

    @functools.partial(
        pl.kernel,
        out_type=jax.ShapeDtypeStruct((n, _EMBED), emb_table.dtype),
        mesh=mesh,
        compiler_params=pltpu.CompilerParams(use_tc_tiling_on_sc=False),
    )
    def gather_kernel(tbl_hbm, idx_hbm, out_hbm):
        def body(i_vmem, o_vmem):
            pltpu.sync_copy(tbl_hbm.at[i_vmem.at[0]], o_vmem)

        pltpu.emit_pipeline(
            body,
            grid=(n // _GATHER_WINDOW,),
            in_specs=[
                pl.BlockSpec((1, _GATHER_WINDOW), index_map=lambda i: (0, i))
            ],
            out_specs=[
                pl.BlockSpec((_GATHER_WINDOW, _EMBED), index_map=lambda i: (i, 0))
            ],
            core_axis_name=("c", "s"),
            dimension_semantics=(pltpu.PARALLEL,),
        )(idx_hbm, out_hbm)

    return gather_kernel(emb_table, flat_ids)


def _tc_mlp_body(emb_ref, meta_ref, w1a_ref, w1b_ref, b1_ref, w2_ref, b2_ref,
                 gamma_ref, beta_ref, out_ref):
    p, nb = emb_ref.shape[0], emb_ref.shape[1]
    emb = emb_ref[...].reshape(p * nb, _EMBED)
    meta = meta_ref[...].reshape(p * nb, _NMETA)
    h = (
        jnp.dot(emb, w1a_ref[...], preferred_element_type=jnp.float32)
        + jnp.dot(meta, w1b_ref[...], preferred_element_type=jnp.float32)
        + b1_ref[...]
    )
    h = jnp.maximum(h, 0.0)
    h = jnp.dot(h, w2_ref[...], preferred_element_type=jnp.float32) + b2_ref[...]
    mean = jnp.mean(h, axis=-1, keepdims=True)
    d = h - mean
    var = jnp.mean(d * d, axis=-1, keepdims=True)
    y = d * jax.lax.rsqrt(var + 1e-5) * gamma_ref[...] + beta_ref[...]
    out_ref[...] = jnp.max(y.reshape(p, nb, _CTX), axis=0)


def _tc_mlp(embeds3, meta3, w1a, w1b, b1, w2, b2, gamma, beta):
    p, bsz = embeds3.shape[0], embeds3.shape[1]
    fixed = lambda i: (0, 0)
    return pl.pallas_call(
        _tc_mlp_body,
        grid=(bsz // _BBLK,),
        in_specs=[
            pl.BlockSpec((p, _BBLK, _EMBED), lambda i: (0, i, 0)),
            pl.BlockSpec((p, _BBLK, _NMETA), lambda i: (0, i, 0)),
            pl.BlockSpec((_EMBED, _CTX), fixed),
            pl.BlockSpec((_NMETA, _CTX), fixed),
            pl.BlockSpec((1, _CTX), fixed),
            pl.BlockSpec((_CTX, _CTX), fixed),
            pl.BlockSpec((1, _CTX), fixed),
            pl.BlockSpec((1, _CTX), fixed),
            pl.BlockSpec((1, _CTX), fixed),
        ],
        out_specs=pl.BlockSpec((_BBLK, _CTX), lambda i: (i, 0)),
        out_shape=jax.ShapeDtypeStruct((bsz, _CTX), jnp.float32),
    )(embeds3, meta3, w1a, w1b, b1, w2, b2, gamma, beta)


_NCHUNK = 4


def kernel(pattern_ids, pattern_metadata, emb_table, W1, b1, W2, b2, gamma, beta):
    bsz, p = pattern_ids.shape
    w1a = W1[:_EMBED]
    w1b = W1[_EMBED:]
    b1r, b2r = b1.reshape(1, _CTX), b2.reshape(1, _CTX)
    gr, br = gamma.reshape(1, _CTX), beta.reshape(1, _CTX)
    cb = bsz // _NCHUNK
    outs = []
    for c in range(_NCHUNK):
        ids_c = pattern_ids[c * cb:(c + 1) * cb]
        meta_c = pattern_metadata[c * cb:(c + 1) * cb]
        flat_ids = ids_c.T.reshape(1, cb * p).astype(jnp.int32)
        embeds = _sc_gather(emb_table, flat_ids)
        outs.append(_tc_mlp(
            embeds.reshape(p, cb, _EMBED), meta_c.transpose(1, 0, 2),
            w1a, w1b, b1r, W2, b2r, gr, br,
        ))
    return jnp.concatenate(outs, axis=0)
